# baseline (speedup 1.0000x reference)
import functools

import jax
import jax.numpy as jnp
from jax import lax
from jax.experimental import pallas as pl
from jax.experimental.pallas import tpu as pltpu
from jax.experimental.pallas import tpu_sc as plsc

L = 16
NC = 2
NS = 16
TM = 128


_INTERPRET = False


def _bc(s):
  return lax.broadcast_in_dim(s, (L,), ())


def _routing_kernel(T, P, H, E, NPAD, NT):
  CP = P // NS
  CT = T // NS
  TCH = 32
  K = P // T

  mesh = plsc.VectorSubcoreMesh(
      core_axis_name="c", subcore_axis_name="s", num_cores=NC, num_subcores=NS)

  @functools.partial(
      pl.kernel,
      out_type=(
          jax.ShapeDtypeStruct((NPAD, H), jnp.float32),
          jax.ShapeDtypeStruct((T,), jnp.int32),
          jax.ShapeDtypeStruct((T,), jnp.int32),
          jax.ShapeDtypeStruct((NT,), jnp.int32),
      ),
      mesh=mesh,
      scratch_types=dict(
          idx_v=pltpu.VMEM((CP,), jnp.int32),
          pos_v=pltpu.VMEM((CP,), jnp.int32),
          base_v=pltpu.VMEM((L,), jnp.int32),
          cnt_v=pltpu.VMEM((L,), jnp.int32),
          hist_sh=pltpu.VMEM_SHARED((NS * L,), jnp.int32),
          hist_v=pltpu.VMEM((NS * L,), jnp.int32),
          rows_v=pltpu.VMEM((TCH, H), jnp.float32),
          eidx_v=pltpu.VMEM((TCH,), jnp.int32),
          oidx_v=pltpu.VMEM((TCH,), jnp.int32),
          iev_v=pltpu.VMEM((CT,), jnp.int32),
          iov_v=pltpu.VMEM((CT,), jnp.int32),
          te_v=pltpu.VMEM((NT,), jnp.int32),
          sem=pltpu.SemaphoreType.DMA,
      ),
      compiler_params=pltpu.CompilerParams(needs_layout_passes=False),
      interpret=_INTERPRET,
  )
  def routing(hidden_hbm, fidx_hbm, ei_hbm, iev_hbm, iov_hbm, te_hbm,
              idx_v, pos_v, base_v, cnt_v, hist_sh, hist_v, rows_v,
              eidx_v, oidx_v, iev_v, iov_v, te_v, sem):
    cid = lax.axis_index("c")
    wid = lax.axis_index("s")

    @pl.when(cid == 0)
    def _():
      iota = lax.iota(jnp.int32, L)

      zv = jnp.zeros((L,), jnp.int32)

      ov = jnp.ones((L,), jnp.int32)

      pltpu.sync_copy(fidx_hbm.at[pl.ds(wid * CP, CP)], idx_v)
      cnt = zv
      for v in range(CP // L):
        ids = idx_v[pl.ds(v * L, L)]
        for e in range(E):
          s = jnp.sum(jnp.where(ids == e, ov, zv))
          cnt = cnt + jnp.where(iota == e, _bc(s), zv)
      cnt_v[...] = cnt
      pltpu.sync_copy(cnt_v, hist_sh.at[pl.ds(wid * L, L)])
      plsc.subcore_barrier()

      pltpu.sync_copy(hist_sh, hist_v)
      totals = zv
      prefix = zv
      for w in range(NS):
        row = hist_v[pl.ds(w * L, L)]
        totals = totals + row
        wlt = jnp.where(jnp.int32(w) < wid, jnp.int32(1), jnp.int32(0))
        prefix = prefix + row * _bc(wlt)
      padded = jnp.bitwise_and(totals + (TM - 1), -TM)
      incl = plsc.cumsum(padded)
      base = (incl - padded) + prefix

      for v in range(CP // L):
        ids = idx_v[pl.ds(v * L, L)]
        rank = zv
        cnt = zv
        for m in range(L):
          bm = jnp.sum(jnp.where(iota == m, ids, zv))
          b = _bc(bm)
          rank = rank + jnp.where((ids == b) & (iota > m), ov, zv)
          cnt = cnt + jnp.where(iota == b, ov, zv)
        gbase = zv
        for e in range(E):
          s = jnp.sum(jnp.where(iota == e, base, zv))
          gbase = gbase + jnp.where(ids == e, _bc(s), zv)
        pos_v[pl.ds(v * L, L)] = gbase + rank
        base = base + cnt

      for u in range(CT // L):
        ev = plsc.load_gather(pos_v, [iota * K + u * (K * L)])
        od = plsc.load_gather(pos_v, [iota * K + u * (K * L) + 1])
        iev_v[pl.ds(u * L, L)] = ev
        iov_v[pl.ds(u * L, L)] = od
      pltpu.sync_copy(iev_v, iev_hbm.at[pl.ds(wid * CT, CT)])
      pltpu.sync_copy(iov_v, iov_hbm.at[pl.ds(wid * CT, CT)])

      for ch in range(CT // TCH):
        t0 = ch * TCH
        pltpu.sync_copy(hidden_hbm.at[pl.ds(wid * CT + t0, TCH)], rows_v)
        for u in range(TCH // L):
          eidx_v[pl.ds(u * L, L)] = iev_v[pl.ds(t0 + u * L, L)]
          oidx_v[pl.ds(u * L, L)] = iov_v[pl.ds(t0 + u * L, L)]
        d1 = pltpu.async_copy(rows_v, ei_hbm.at[eidx_v], sem)
        d2 = pltpu.async_copy(rows_v, ei_hbm.at[oidx_v], sem)
        d1.wait()
        d2.wait()

      @pl.when(wid == 0)
      def _():
        for g in range(NT // L):
          trow = (iota + g * L) * TM
          acc = zv
          for e in range(E):
            s = jnp.sum(jnp.where(iota == e, incl, zv))
            acc = acc + jnp.where(trow >= _bc(s), ov, zv)
          te_v[pl.ds(g * L, L)] = jnp.minimum(acc, E - 1)
        pltpu.sync_copy(te_v, te_hbm)

  return routing


def _gemm_body(te_ref, x_ref, gw_ref, uw_ref, dw_ref, o_ref):
  x = x_ref[...].astype(jnp.bfloat16)
  g = lax.dot_general(x, gw_ref[0].astype(jnp.bfloat16),
                      (((1,), (1,)), ((), ())),
                      preferred_element_type=jnp.float32)
  u = lax.dot_general(x, uw_ref[0].astype(jnp.bfloat16),
                      (((1,), (1,)), ((), ())),
                      preferred_element_type=jnp.float32)
  h = (g * jax.nn.sigmoid(g) * u).astype(jnp.bfloat16)
  o_ref[...] = lax.dot_general(h, dw_ref[0].astype(jnp.bfloat16),
                               (((1,), (1,)), ((), ())),
                               preferred_element_type=jnp.float32)


def _combine_kernel(T, P, H, NPAD):
  NW = NC * NS
  TW = T // NW
  CW = 32
  K = P // T

  mesh = plsc.VectorSubcoreMesh(
      core_axis_name="c", subcore_axis_name="s", num_cores=NC, num_subcores=NS)

  @functools.partial(
      pl.kernel,
      out_type=jax.ShapeDtypeStruct((T, H), jnp.float32),
      mesh=mesh,
      scratch_types=dict(
          ie_v=pltpu.VMEM((TW,), jnp.int32),
          io_v=pltpu.VMEM((TW,), jnp.int32),
          iec_v=pltpu.VMEM((CW,), jnp.int32),
          ioc_v=pltpu.VMEM((CW,), jnp.int32),
          w_v=pltpu.VMEM((K * TW,), jnp.float32),
          bufE=pltpu.VMEM((CW, H), jnp.float32),
          bufO=pltpu.VMEM((CW, H), jnp.float32),
          sem=pltpu.SemaphoreType.DMA,
      ),
      compiler_params=pltpu.CompilerParams(needs_layout_passes=False),
      interpret=_INTERPRET,
  )
  def combine(y_hbm, iev_hbm, iov_hbm, w_hbm, out_hbm,
              ie_v, io_v, iec_v, ioc_v, w_v, bufE, bufO, sem):
    cid = lax.axis_index("c")
    sid = lax.axis_index("s")
    wid = cid * NS + sid

    pltpu.sync_copy(iev_hbm.at[pl.ds(wid * TW, TW)], ie_v)
    pltpu.sync_copy(iov_hbm.at[pl.ds(wid * TW, TW)], io_v)
    pltpu.sync_copy(w_hbm.at[pl.ds(wid * TW * K, TW * K)], w_v)

    for ch in range(TW // CW):
      for u in range(CW // L):
        iec_v[pl.ds(u * L, L)] = ie_v[pl.ds(ch * CW + u * L, L)]
        ioc_v[pl.ds(u * L, L)] = io_v[pl.ds(ch * CW + u * L, L)]
      d1 = pltpu.async_copy(y_hbm.at[iec_v], bufE, sem)
      d2 = pltpu.async_copy(y_hbm.at[ioc_v], bufO, sem)
      d1.wait()
      d2.wait()

      def body(t, carry, ch=ch):
        tl = ch * CW + t
        we = plsc.load_gather(w_v, [jnp.full((L,), K * tl, jnp.int32)])
        wo = plsc.load_gather(w_v, [jnp.full((L,), K * tl + 1, jnp.int32)])
        for j in range(H // L):
          ev = bufE[t, pl.ds(j * L, L)]
          ov = bufO[t, pl.ds(j * L, L)]
          bufE[t, pl.ds(j * L, L)] = ev * we + ov * wo
        return carry

      lax.fori_loop(0, CW, body, jnp.int32(0))
      pltpu.sync_copy(bufE, out_hbm.at[pl.ds(wid * TW + ch * CW, CW)])

  return combine


def kernel(hidden_states, topk_idx, topk_weight, gate_weight, up_weight,
           down_weight):
  B, S, H = hidden_states.shape
  E, F, _ = gate_weight.shape
  T = B * S
  K = topk_idx.shape[-1]
  P = T * K
  NPAD = P + E * TM
  NT = NPAD // TM

  hidden_flat = hidden_states.reshape(T, H)
  flat_idx = topk_idx.reshape(-1)
  flat_w = topk_weight.reshape(-1)

  routing = _routing_kernel(T, P, H, E, NPAD, NT)
  ei, idx_even, idx_odd, tile_expert = routing(hidden_flat, flat_idx)

  y = pl.pallas_call(
      _gemm_body,
      grid_spec=pltpu.PrefetchScalarGridSpec(
          num_scalar_prefetch=1,
          grid=(NT,),
          in_specs=[
              pl.BlockSpec((TM, H), lambda i, te: (i, 0)),
              pl.BlockSpec((1, F, H), lambda i, te: (te[i], 0, 0)),
              pl.BlockSpec((1, F, H), lambda i, te: (te[i], 0, 0)),
              pl.BlockSpec((1, H, F), lambda i, te: (te[i], 0, 0)),
          ],
          out_specs=pl.BlockSpec((TM, H), lambda i, te: (i, 0)),
      ),
      out_shape=jax.ShapeDtypeStruct((NPAD, H), jnp.float32),
      compiler_params=pltpu.CompilerParams(
          dimension_semantics=("arbitrary",),
          vmem_limit_bytes=100 * 1024 * 1024,
      ),
      interpret=_INTERPRET,
  )(tile_expert, ei, gate_weight, up_weight, down_weight)

  combine = _combine_kernel(T, P, H, NPAD)
  out_flat = combine(y, idx_even, idx_odd, flat_w)
  return out_flat.reshape(B, S, H)


# speedup vs baseline: 1.0595x; 1.0595x over previous
import functools

import jax
import jax.numpy as jnp
from jax import lax
from jax.experimental import pallas as pl
from jax.experimental.pallas import tpu as pltpu
from jax.experimental.pallas import tpu_sc as plsc

L = 16
NC = 2
NS = 16
TM = 128


_INTERPRET = False


def _bc(s):
  return lax.broadcast_in_dim(s, (L,), ())


def _routing_kernel(T, P, H, E, NPAD, NT):
  CP = P // NS
  CT = T // NS
  TCH = 32
  K = P // T

  mesh = plsc.VectorSubcoreMesh(
      core_axis_name="c", subcore_axis_name="s", num_cores=NC, num_subcores=NS)

  @functools.partial(
      pl.kernel,
      out_type=(
          jax.ShapeDtypeStruct((NPAD, H), jnp.float32),
          jax.ShapeDtypeStruct((T,), jnp.int32),
          jax.ShapeDtypeStruct((T,), jnp.int32),
          jax.ShapeDtypeStruct((NT,), jnp.int32),
      ),
      mesh=mesh,
      scratch_types=dict(
          idx_v=pltpu.VMEM((CP,), jnp.int32),
          pos_v=pltpu.VMEM((CP,), jnp.int32),
          base_v=pltpu.VMEM((L,), jnp.int32),
          cnt_v=pltpu.VMEM((L,), jnp.int32),
          hist_sh=pltpu.VMEM_SHARED((NS * L,), jnp.int32),
          hist_v=pltpu.VMEM((NS * L,), jnp.int32),
          rows_v=pltpu.VMEM((TCH, H), jnp.float32),
          eidx_v=pltpu.VMEM((TCH,), jnp.int32),
          oidx_v=pltpu.VMEM((TCH,), jnp.int32),
          iev_v=pltpu.VMEM((CT,), jnp.int32),
          iov_v=pltpu.VMEM((CT,), jnp.int32),
          te_v=pltpu.VMEM((NT,), jnp.int32),
          sem=pltpu.SemaphoreType.DMA,
      ),
      compiler_params=pltpu.CompilerParams(needs_layout_passes=False),
      interpret=_INTERPRET,
  )
  def routing(hidden_hbm, fidx_hbm, ei_hbm, iev_hbm, iov_hbm, te_hbm,
              idx_v, pos_v, base_v, cnt_v, hist_sh, hist_v, rows_v,
              eidx_v, oidx_v, iev_v, iov_v, te_v, sem):
    cid = lax.axis_index("c")
    wid = lax.axis_index("s")
    HT = CT // 2

    iota = lax.iota(jnp.int32, L)
    zv = jnp.zeros((L,), jnp.int32)
    ov = jnp.ones((L,), jnp.int32)

    pltpu.sync_copy(fidx_hbm.at[pl.ds(wid * CP, CP)], idx_v)
    cnt = zv
    for v in range(CP // L):
      ids = idx_v[pl.ds(v * L, L)]
      for e in range(E):
        s = jnp.sum(jnp.where(ids == e, ov, zv))
        cnt = cnt + jnp.where(iota == e, _bc(s), zv)
    cnt_v[...] = cnt
    pltpu.sync_copy(cnt_v, hist_sh.at[pl.ds(wid * L, L)])
    plsc.subcore_barrier()

    pltpu.sync_copy(hist_sh, hist_v)
    totals = zv
    prefix = zv
    for w in range(NS):
      row = hist_v[pl.ds(w * L, L)]
      totals = totals + row
      wlt = jnp.where(jnp.int32(w) < wid, jnp.int32(1), jnp.int32(0))
      prefix = prefix + row * _bc(wlt)
    padded = jnp.bitwise_and(totals + (TM - 1), -TM)
    incl = plsc.cumsum(padded)
    base = (incl - padded) + prefix

    for v in range(CP // L):
      ids = idx_v[pl.ds(v * L, L)]
      rank = zv
      cnt = zv
      for m in range(L):
        bm = jnp.sum(jnp.where(iota == m, ids, zv))
        b = _bc(bm)
        rank = rank + jnp.where((ids == b) & (iota > m), ov, zv)
        cnt = cnt + jnp.where(iota == b, ov, zv)
      gbase = zv
      for e in range(E):
        s = jnp.sum(jnp.where(iota == e, base, zv))
        gbase = gbase + jnp.where(ids == e, _bc(s), zv)
      pos_v[pl.ds(v * L, L)] = gbase + rank
      base = base + cnt

    for u in range(CT // L):
      ev = plsc.load_gather(pos_v, [iota * K + u * (K * L)])
      od = plsc.load_gather(pos_v, [iota * K + u * (K * L) + 1])
      iev_v[pl.ds(u * L, L)] = ev
      iov_v[pl.ds(u * L, L)] = od
    off = cid * HT
    pltpu.sync_copy(iev_v.at[pl.ds(off, HT)],
                    iev_hbm.at[pl.ds(wid * CT + off, HT)])
    pltpu.sync_copy(iov_v.at[pl.ds(off, HT)],
                    iov_hbm.at[pl.ds(wid * CT + off, HT)])

    for ch in range(HT // TCH):
      t0 = ch * TCH
      pltpu.sync_copy(
          hidden_hbm.at[pl.ds(wid * CT + off + t0, TCH)], rows_v)
      for u in range(TCH // L):
        eidx_v[pl.ds(u * L, L)] = iev_v[pl.ds(off + t0 + u * L, L)]
        oidx_v[pl.ds(u * L, L)] = iov_v[pl.ds(off + t0 + u * L, L)]
      d1 = pltpu.async_copy(rows_v, ei_hbm.at[eidx_v], sem)
      d2 = pltpu.async_copy(rows_v, ei_hbm.at[oidx_v], sem)
      d1.wait()
      d2.wait()

    @pl.when((cid == 0) & (wid == 0))
    def _():
      for g in range(NT // L):
        trow = (iota + g * L) * TM
        acc = zv
        for e in range(E):
          s = jnp.sum(jnp.where(iota == e, incl, zv))
          acc = acc + jnp.where(trow >= _bc(s), ov, zv)
        te_v[pl.ds(g * L, L)] = jnp.minimum(acc, E - 1)
      pltpu.sync_copy(te_v, te_hbm)

  return routing


def _gemm_body(te_ref, x_ref, gw_ref, uw_ref, dw_ref, o_ref):
  x = x_ref[...]
  g = lax.dot_general(x, gw_ref[0], (((1,), (1,)), ((), ())),
                      preferred_element_type=jnp.float32)
  u = lax.dot_general(x, uw_ref[0], (((1,), (1,)), ((), ())),
                      preferred_element_type=jnp.float32)
  h = g * jax.nn.sigmoid(g) * u
  o_ref[...] = lax.dot_general(h, dw_ref[0], (((1,), (1,)), ((), ())),
                               preferred_element_type=jnp.float32)


def _combine_kernel(T, P, H, NPAD):
  NW = NC * NS
  TW = T // NW
  CW = 16
  NCH = TW // CW
  K = P // T

  mesh = plsc.VectorSubcoreMesh(
      core_axis_name="c", subcore_axis_name="s", num_cores=NC, num_subcores=NS)

  @functools.partial(
      pl.kernel,
      out_type=jax.ShapeDtypeStruct((T, H), jnp.float32),
      mesh=mesh,
      scratch_types=dict(
          ie_v=pltpu.VMEM((TW,), jnp.int32),
          io_v=pltpu.VMEM((TW,), jnp.int32),
          iec0=pltpu.VMEM((CW,), jnp.int32),
          iec1=pltpu.VMEM((CW,), jnp.int32),
          ioc0=pltpu.VMEM((CW,), jnp.int32),
          ioc1=pltpu.VMEM((CW,), jnp.int32),
          w_v=pltpu.VMEM((K * TW,), jnp.float32),
          bufE0=pltpu.VMEM((CW, H), jnp.float32),
          bufE1=pltpu.VMEM((CW, H), jnp.float32),
          bufO0=pltpu.VMEM((CW, H), jnp.float32),
          bufO1=pltpu.VMEM((CW, H), jnp.float32),
          sem=pltpu.SemaphoreType.DMA,
          semw=pltpu.SemaphoreType.DMA,
      ),
      compiler_params=pltpu.CompilerParams(needs_layout_passes=False),
      interpret=_INTERPRET,
  )
  def combine(y_hbm, iev_hbm, iov_hbm, w_hbm, out_hbm,
              ie_v, io_v, iec0, iec1, ioc0, ioc1, w_v,
              bufE0, bufE1, bufO0, bufO1, sem, semw):
    cid = lax.axis_index("c")
    sid = lax.axis_index("s")
    wid = cid * NS + sid

    pltpu.sync_copy(iev_hbm.at[pl.ds(wid * TW, TW)], ie_v)
    pltpu.sync_copy(iov_hbm.at[pl.ds(wid * TW, TW)], io_v)
    pltpu.sync_copy(w_hbm.at[pl.ds(wid * TW * K, TW * K)], w_v)

    bufsE = (bufE0, bufE1)
    bufsO = (bufO0, bufO1)
    iecs = (iec0, iec1)
    iocs = (ioc0, ioc1)

    def fill_idx(c):
      p = c & 1
      iecs[p][...] = ie_v[pl.ds(c * CW, CW)]
      iocs[p][...] = io_v[pl.ds(c * CW, CW)]

    def issue_gather(c):
      p = c & 1
      return (pltpu.async_copy(y_hbm.at[iecs[p]], bufsE[p], sem),
              pltpu.async_copy(y_hbm.at[iocs[p]], bufsO[p], sem))

    fill_idx(0)
    gathers = {0: issue_gather(0)}
    pending_w = [None, None]
    for c in range(NCH):
      p = c & 1
      if c + 1 < NCH:
        if pending_w[1 - p] is not None:
          pending_w[1 - p].wait()
          pending_w[1 - p] = None
        fill_idx(c + 1)
        gathers[c + 1] = issue_gather(c + 1)
      d1, d2 = gathers.pop(c)
      d1.wait()
      d2.wait()
      bE, bO = bufsE[p], bufsO[p]

      def body(t, carry, c=c, bE=bE, bO=bO):
        tl = c * CW + t
        we = plsc.load_gather(w_v, [jnp.full((L,), K * tl, jnp.int32)])
        wo = plsc.load_gather(w_v, [jnp.full((L,), K * tl + 1, jnp.int32)])
        for j in range(H // L):
          ev = bE[t, pl.ds(j * L, L)]
          ov = bO[t, pl.ds(j * L, L)]
          bE[t, pl.ds(j * L, L)] = ev * we + ov * wo
        return carry

      lax.fori_loop(0, CW, body, jnp.int32(0))
      pending_w[p] = pltpu.async_copy(
          bE, out_hbm.at[pl.ds(wid * TW + c * CW, CW)], semw)
    for p in (0, 1):
      if pending_w[p] is not None:
        pending_w[p].wait()

  return combine


def kernel(hidden_states, topk_idx, topk_weight, gate_weight, up_weight,
           down_weight):
  B, S, H = hidden_states.shape
  E, F, _ = gate_weight.shape
  T = B * S
  K = topk_idx.shape[-1]
  P = T * K
  NPAD = P + E * TM
  NT = NPAD // TM

  hidden_flat = hidden_states.reshape(T, H)
  flat_idx = topk_idx.reshape(-1)
  flat_w = topk_weight.reshape(-1)

  routing = _routing_kernel(T, P, H, E, NPAD, NT)
  ei, idx_even, idx_odd, tile_expert = routing(hidden_flat, flat_idx)

  y = pl.pallas_call(
      _gemm_body,
      grid_spec=pltpu.PrefetchScalarGridSpec(
          num_scalar_prefetch=1,
          grid=(NT,),
          in_specs=[
              pl.BlockSpec((TM, H), lambda i, te: (i, 0)),
              pl.BlockSpec((1, F, H), lambda i, te: (te[i], 0, 0)),
              pl.BlockSpec((1, F, H), lambda i, te: (te[i], 0, 0)),
              pl.BlockSpec((1, H, F), lambda i, te: (te[i], 0, 0)),
          ],
          out_specs=pl.BlockSpec((TM, H), lambda i, te: (i, 0)),
      ),
      out_shape=jax.ShapeDtypeStruct((NPAD, H), jnp.float32),
      compiler_params=pltpu.CompilerParams(
          dimension_semantics=("arbitrary",),
          vmem_limit_bytes=100 * 1024 * 1024,
      ),
      interpret=_INTERPRET,
  )(tile_expert, ei, gate_weight, up_weight, down_weight)

  combine = _combine_kernel(T, P, H, NPAD)
  out_flat = combine(y, idx_even, idx_odd, flat_w)
  return out_flat.reshape(B, S, H)


# speedup vs baseline: 1.0614x; 1.0018x over previous
import functools

import jax
import jax.numpy as jnp
from jax import lax
from jax.experimental import pallas as pl
from jax.experimental.pallas import tpu as pltpu
from jax.experimental.pallas import tpu_sc as plsc

L = 16
NC = 2
NS = 16
TM = 128


def _bc(s):
  return lax.broadcast_in_dim(s, (L,), ())


def _routing_kernel(T, P, H, E, NPAD, NT):
  CP = P // NS
  CT = T // NS
  TCH = 32
  K = P // T

  mesh = plsc.VectorSubcoreMesh(
      core_axis_name="c", subcore_axis_name="s", num_cores=NC, num_subcores=NS)

  @functools.partial(
      pl.kernel,
      out_type=(
          jax.ShapeDtypeStruct((NPAD, H), jnp.float32),
          jax.ShapeDtypeStruct((T,), jnp.int32),
          jax.ShapeDtypeStruct((T,), jnp.int32),
          jax.ShapeDtypeStruct((NT,), jnp.int32),
      ),
      mesh=mesh,
      scratch_types=dict(
          idx_v=pltpu.VMEM((CP,), jnp.int32),
          pos_v=pltpu.VMEM((CP,), jnp.int32),
          base_v=pltpu.VMEM((L,), jnp.int32),
          cnt_v=pltpu.VMEM((L,), jnp.int32),
          hist_sh=pltpu.VMEM_SHARED((NS * L,), jnp.int32),
          hist_v=pltpu.VMEM((NS * L,), jnp.int32),
          rows_v=pltpu.VMEM((TCH, H), jnp.float32),
          eidx_v=pltpu.VMEM((TCH,), jnp.int32),
          oidx_v=pltpu.VMEM((TCH,), jnp.int32),
          iev_v=pltpu.VMEM((CT,), jnp.int32),
          iov_v=pltpu.VMEM((CT,), jnp.int32),
          te_v=pltpu.VMEM((NT,), jnp.int32),
          sem=pltpu.SemaphoreType.DMA,
      ),
      compiler_params=pltpu.CompilerParams(needs_layout_passes=False),
  )
  def routing(hidden_hbm, fidx_hbm, ei_hbm, iev_hbm, iov_hbm, te_hbm,
              idx_v, pos_v, base_v, cnt_v, hist_sh, hist_v, rows_v,
              eidx_v, oidx_v, iev_v, iov_v, te_v, sem):
    cid = lax.axis_index("c")
    wid = lax.axis_index("s")
    HT = CT // 2

    iota = lax.iota(jnp.int32, L)
    zv = jnp.zeros((L,), jnp.int32)
    ov = jnp.ones((L,), jnp.int32)

    pltpu.sync_copy(fidx_hbm.at[pl.ds(wid * CP, CP)], idx_v)
    cnt = zv
    for v in range(CP // L):
      ids = idx_v[pl.ds(v * L, L)]
      for e in range(E):
        s = jnp.sum(jnp.where(ids == e, ov, zv))
        cnt = cnt + jnp.where(iota == e, _bc(s), zv)
    cnt_v[...] = cnt
    pltpu.sync_copy(cnt_v, hist_sh.at[pl.ds(wid * L, L)])
    plsc.subcore_barrier()

    pltpu.sync_copy(hist_sh, hist_v)
    totals = zv
    prefix = zv
    for w in range(NS):
      row = hist_v[pl.ds(w * L, L)]
      totals = totals + row
      wlt = jnp.where(jnp.int32(w) < wid, jnp.int32(1), jnp.int32(0))
      prefix = prefix + row * _bc(wlt)
    padded = jnp.bitwise_and(totals + (TM - 1), -TM)
    incl = plsc.cumsum(padded)
    base = (incl - padded) + prefix

    for v in range(CP // L):
      ids = idx_v[pl.ds(v * L, L)]
      rank = zv
      cnt = zv
      for m in range(L):
        bm = jnp.sum(jnp.where(iota == m, ids, zv))
        b = _bc(bm)
        rank = rank + jnp.where((ids == b) & (iota > m), ov, zv)
        cnt = cnt + jnp.where(iota == b, ov, zv)
      gbase = zv
      for e in range(E):
        s = jnp.sum(jnp.where(iota == e, base, zv))
        gbase = gbase + jnp.where(ids == e, _bc(s), zv)
      pos_v[pl.ds(v * L, L)] = gbase + rank
      base = base + cnt

    for u in range(CT // L):
      ev = plsc.load_gather(pos_v, [iota * K + u * (K * L)])
      od = plsc.load_gather(pos_v, [iota * K + u * (K * L) + 1])
      iev_v[pl.ds(u * L, L)] = ev
      iov_v[pl.ds(u * L, L)] = od
    off = cid * HT
    pltpu.sync_copy(iev_v.at[pl.ds(off, HT)],
                    iev_hbm.at[pl.ds(wid * CT + off, HT)])
    pltpu.sync_copy(iov_v.at[pl.ds(off, HT)],
                    iov_hbm.at[pl.ds(wid * CT + off, HT)])

    for ch in range(HT // TCH):
      t0 = ch * TCH
      pltpu.sync_copy(
          hidden_hbm.at[pl.ds(wid * CT + off + t0, TCH)], rows_v)
      for u in range(TCH // L):
        eidx_v[pl.ds(u * L, L)] = iev_v[pl.ds(off + t0 + u * L, L)]
        oidx_v[pl.ds(u * L, L)] = iov_v[pl.ds(off + t0 + u * L, L)]
      d1 = pltpu.async_copy(rows_v, ei_hbm.at[eidx_v], sem)
      d2 = pltpu.async_copy(rows_v, ei_hbm.at[oidx_v], sem)
      d1.wait()
      d2.wait()

    @pl.when((cid == 0) & (wid == 0))
    def _():
      for g in range(NT // L):
        trow = (iota + g * L) * TM
        acc = zv
        for e in range(E):
          s = jnp.sum(jnp.where(iota == e, incl, zv))
          acc = acc + jnp.where(trow >= _bc(s), ov, zv)
        te_v[pl.ds(g * L, L)] = jnp.minimum(acc, E - 1)
      pltpu.sync_copy(te_v, te_hbm)

  return routing


def _gemm_body(te_ref, x_ref, gw_ref, uw_ref, dw_ref, o_ref):
  x = x_ref[...]
  g = lax.dot_general(x, gw_ref[0], (((1,), (1,)), ((), ())),
                      preferred_element_type=jnp.float32)
  u = lax.dot_general(x, uw_ref[0], (((1,), (1,)), ((), ())),
                      preferred_element_type=jnp.float32)
  h = g * jax.nn.sigmoid(g) * u
  o_ref[...] = lax.dot_general(h, dw_ref[0], (((1,), (1,)), ((), ())),
                               preferred_element_type=jnp.float32)


def _combine_kernel(T, P, H, NPAD):
  NW = NC * NS
  TW = T // NW
  CW = 16
  NCH = TW // CW
  K = P // T

  mesh = plsc.VectorSubcoreMesh(
      core_axis_name="c", subcore_axis_name="s", num_cores=NC, num_subcores=NS)

  @functools.partial(
      pl.kernel,
      out_type=jax.ShapeDtypeStruct((T, H), jnp.float32),
      mesh=mesh,
      scratch_types=dict(
          ie_v=pltpu.VMEM((TW,), jnp.int32),
          io_v=pltpu.VMEM((TW,), jnp.int32),
          iec0=pltpu.VMEM((CW,), jnp.int32),
          iec1=pltpu.VMEM((CW,), jnp.int32),
          ioc0=pltpu.VMEM((CW,), jnp.int32),
          ioc1=pltpu.VMEM((CW,), jnp.int32),
          w_v=pltpu.VMEM((K * TW,), jnp.float32),
          bufE0=pltpu.VMEM((CW, H), jnp.float32),
          bufE1=pltpu.VMEM((CW, H), jnp.float32),
          bufO0=pltpu.VMEM((CW, H), jnp.float32),
          bufO1=pltpu.VMEM((CW, H), jnp.float32),
          sem=pltpu.SemaphoreType.DMA,
          semw=pltpu.SemaphoreType.DMA,
      ),
      compiler_params=pltpu.CompilerParams(needs_layout_passes=False),
  )
  def combine(y_hbm, iev_hbm, iov_hbm, w_hbm, out_hbm,
              ie_v, io_v, iec0, iec1, ioc0, ioc1, w_v,
              bufE0, bufE1, bufO0, bufO1, sem, semw):
    cid = lax.axis_index("c")
    sid = lax.axis_index("s")
    wid = cid * NS + sid

    pltpu.sync_copy(iev_hbm.at[pl.ds(wid * TW, TW)], ie_v)
    pltpu.sync_copy(iov_hbm.at[pl.ds(wid * TW, TW)], io_v)
    pltpu.sync_copy(w_hbm.at[pl.ds(wid * TW * K, TW * K)], w_v)

    bufsE = (bufE0, bufE1)
    bufsO = (bufO0, bufO1)
    iecs = (iec0, iec1)
    iocs = (ioc0, ioc1)

    def fill_idx(c):
      p = c & 1
      iecs[p][...] = ie_v[pl.ds(c * CW, CW)]
      iocs[p][...] = io_v[pl.ds(c * CW, CW)]

    def issue_gather(c):
      p = c & 1
      return (pltpu.async_copy(y_hbm.at[iecs[p]], bufsE[p], sem),
              pltpu.async_copy(y_hbm.at[iocs[p]], bufsO[p], sem))

    fill_idx(0)
    gathers = {0: issue_gather(0)}
    pending_w = [None, None]
    for c in range(NCH):
      p = c & 1
      if c + 1 < NCH:
        if pending_w[1 - p] is not None:
          pending_w[1 - p].wait()
          pending_w[1 - p] = None
        fill_idx(c + 1)
        gathers[c + 1] = issue_gather(c + 1)
      d1, d2 = gathers.pop(c)
      d1.wait()
      d2.wait()
      bE, bO = bufsE[p], bufsO[p]

      def body(t, carry, c=c, bE=bE, bO=bO):
        tl = c * CW + t
        we = plsc.load_gather(w_v, [jnp.full((L,), K * tl, jnp.int32)])
        wo = plsc.load_gather(w_v, [jnp.full((L,), K * tl + 1, jnp.int32)])
        for j in range(H // L):
          ev = bE[t, pl.ds(j * L, L)]
          ov = bO[t, pl.ds(j * L, L)]
          bE[t, pl.ds(j * L, L)] = ev * we + ov * wo
        return carry

      lax.fori_loop(0, CW, body, jnp.int32(0))
      pending_w[p] = pltpu.async_copy(
          bE, out_hbm.at[pl.ds(wid * TW + c * CW, CW)], semw)
    for p in (0, 1):
      if pending_w[p] is not None:
        pending_w[p].wait()

  return combine


def kernel(hidden_states, topk_idx, topk_weight, gate_weight, up_weight,
           down_weight):
  B, S, H = hidden_states.shape
  E, F, _ = gate_weight.shape
  T = B * S
  K = topk_idx.shape[-1]
  P = T * K
  NPAD = P + E * TM
  NT = NPAD // TM

  hidden_flat = hidden_states.reshape(T, H)
  flat_idx = topk_idx.reshape(-1)
  flat_w = topk_weight.reshape(-1)

  routing = _routing_kernel(T, P, H, E, NPAD, NT)
  ei, idx_even, idx_odd, tile_expert = routing(hidden_flat, flat_idx)

  y = pl.pallas_call(
      _gemm_body,
      grid_spec=pltpu.PrefetchScalarGridSpec(
          num_scalar_prefetch=1,
          grid=(NT,),
          in_specs=[
              pl.BlockSpec((TM, H), lambda i, te: (i, 0)),
              pl.BlockSpec((1, F, H), lambda i, te: (te[i], 0, 0)),
              pl.BlockSpec((1, F, H), lambda i, te: (te[i], 0, 0)),
              pl.BlockSpec((1, H, F), lambda i, te: (te[i], 0, 0)),
          ],
          out_specs=pl.BlockSpec((TM, H), lambda i, te: (i, 0)),
      ),
      out_shape=jax.ShapeDtypeStruct((NPAD, H), jnp.float32),
      compiler_params=pltpu.CompilerParams(
          dimension_semantics=("arbitrary",),
          vmem_limit_bytes=100 * 1024 * 1024,
      ),
  )(tile_expert, ei, gate_weight, up_weight, down_weight)

  combine = _combine_kernel(T, P, H, NPAD)
  out_flat = combine(y, idx_even, idx_odd, flat_w)
  return out_flat.reshape(B, S, H)
